# baseline (speedup 1.0000x reference)
import jax
import jax.numpy as jnp
from jax import lax
from jax.experimental import pallas as pl
from jax.experimental.pallas import tpu as pltpu
from jax.experimental.pallas import tpu_sc as plsc

B = 16
N = 131072
SH = 64
NBINS = SH * SH * SH
Q = 4
QB = NBINS // Q
PCH = 4096
NCH = N // PCH
XCH = 8192

NC = 2
NS = 16
NW = NC * NS


def _sc_body(z_hbm, x_hbm, counts_hbm, r_hbm, zbuf0, zbuf1, hist, xbuf,
             rbuf, sem0, sem1):
    c = lax.axis_index("c")
    s = lax.axis_index("s")
    wid = s * NC + c
    lane3 = lax.iota(jnp.int32, 16) * 3
    ones = jnp.ones((16,), jnp.float32)
    sems = (sem0, sem1)
    zbufs = (zbuf0, zbuf1)

    def z_src(b, ch):
        return z_hbm.at[b * NCH + ch]

    for rep in range(2):
        pair = wid + rep * NW
        b = pair // Q
        q = pair % Q
        qlo = q * QB

        @pl.loop(0, QB // 16, unroll=8)
        def _(i):
            hist[pl.ds(i * 16, 16)] = jnp.zeros((16,), jnp.float32)

        pltpu.async_copy(z_src(b, 0), zbufs[0], sems[0])

        def hist_chunk(ch, par):
            pltpu.make_async_copy(z_src(b, ch), zbufs[par], sems[par]).wait()

            @pl.when(ch + 1 < NCH)
            def _():
                pltpu.async_copy(
                    z_src(b, ch + 1), zbufs[1 - par], sems[1 - par])

            zrow = zbufs[par]

            @pl.loop(0, PCH // 16, unroll=4)
            def _(i):
                i0 = lane3 + i * 48
                v0 = plsc.load_gather(zrow, [i0])
                v1 = plsc.load_gather(zrow, [i0 + 1])
                v2 = plsc.load_gather(zrow, [i0 + 2])
                w0 = ((v0 * 64.0) * 0.9999999).astype(jnp.int32)
                w1 = ((v1 * 64.0) * 0.9999999).astype(jnp.int32)
                w2 = ((v2 * 64.0) * 0.9999999).astype(jnp.int32)
                lin = (w0 << 12) + (w1 << 6) + w2
                loc = plsc.bitcast(lin - qlo, jnp.uint32)
                m = loc < QB
                locc = plsc.bitcast(jnp.minimum(loc, QB - 1), jnp.int32)
                plsc.addupdate_scatter(hist, [locc], ones, mask=m)

        @pl.loop(0, NCH // 2)
        def _(g):
            for par in range(2):
                hist_chunk(g * 2 + par, par)

        pltpu.sync_copy(hist, counts_hbm.at[pair])
        xrow = pair * (QB // XCH)

        @pl.loop(0, QB // XCH)
        def _(t):
            pltpu.sync_copy(x_hbm.at[xrow + t], xbuf)

            @pl.loop(0, XCH // 16, unroll=4)
            def _(j):
                xv = xbuf[pl.ds(j * 16, 16)]
                hv = hist[pl.ds(t * XCH + j * 16, 16)]
                rbuf[pl.ds(j * 16, 16)] = jnp.where(hv > 0.0, xv, 0.0)

            pltpu.sync_copy(rbuf, r_hbm.at[xrow + t])


@jax.jit
def kernel(x, z):
    xf = x.reshape(B * NBINS // XCH, XCH)
    zf = z.reshape(B * NCH, PCH * 3)
    mesh = plsc.VectorSubcoreMesh(core_axis_name="c", subcore_axis_name="s")
    counts_f, r_f = pl.kernel(
        _sc_body,
        out_type=(
            jax.ShapeDtypeStruct((B * Q, QB), jnp.float32),
            jax.ShapeDtypeStruct((B * NBINS // XCH, XCH), jnp.float32),
        ),
        mesh=mesh,
        compiler_params=pltpu.CompilerParams(needs_layout_passes=False),
        scratch_types=[
            pltpu.VMEM((PCH * 3,), jnp.float32),
            pltpu.VMEM((PCH * 3,), jnp.float32),
            pltpu.VMEM((QB,), jnp.float32),
            pltpu.VMEM((XCH,), jnp.float32),
            pltpu.VMEM((XCH,), jnp.float32),
            pltpu.SemaphoreType.DMA,
            pltpu.SemaphoreType.DMA,
        ],
    )(zf, xf)
    return (counts_f.reshape(B, SH, SH, SH), r_f.reshape(B, SH, SH, SH))


# speedup vs baseline: 18.4942x; 18.4942x over previous
import jax
import jax.numpy as jnp
from jax import lax
from jax.experimental import pallas as pl
from jax.experimental.pallas import tpu as pltpu
from jax.experimental.pallas import tpu_sc as plsc

B = 16
N = 131072
SH = 64
NBINS = SH * SH * SH
Q = 4
QB = NBINS // Q
PCH = 4096
NCH = N // PCH
XCH = 8192

NC = 2
NS = 16
NW = NC * NS


def _sc_body(z_hbm, x_hbm, counts_hbm, r_hbm,
             za0, zb0, zc0, za1, zb1, zc1, hist, xbuf, rbuf, sem0, sem1):
    c = lax.axis_index("c")
    s = lax.axis_index("s")
    wid = s * NC + c
    ones = jnp.ones((16,), jnp.float32)
    sems = (sem0, sem1)
    zbufs = ((za0, zb0, zc0), (za1, zb1, zc1))

    def z_copies(b, ch, par):
        return [
            pltpu.make_async_copy(
                z_hbm.at[k, b, pl.ds(ch * PCH, PCH)], zbufs[par][k], sems[par])
            for k in range(3)
        ]

    def z_start(b, ch, par):
        for cp in z_copies(b, ch, par):
            cp.start()

    def z_wait(b, ch, par):
        for cp in z_copies(b, ch, par):
            cp.wait()

    for rep in range(2):
        pair = wid + rep * NW
        b = pair // Q
        q = pair % Q
        qlo = q * QB

        @pl.loop(0, QB // 16, unroll=8)
        def _(i):
            hist[pl.ds(i * 16, 16)] = jnp.zeros((16,), jnp.float32)

        z_start(b, 0, 0)

        def hist_chunk(ch, par):
            z_wait(b, ch, par)

            @pl.when(ch + 1 < NCH)
            def _():
                z_start(b, ch + 1, 1 - par)

            zb = zbufs[par]

            @pl.loop(0, PCH // 16, unroll=4)
            def _(i):
                sl = pl.ds(i * 16, 16)
                v0 = zb[0][sl]
                v1 = zb[1][sl]
                v2 = zb[2][sl]
                w0 = ((v0 * 64.0) * 0.9999999).astype(jnp.int32)
                w1 = ((v1 * 64.0) * 0.9999999).astype(jnp.int32)
                w2 = ((v2 * 64.0) * 0.9999999).astype(jnp.int32)
                lin = (w0 << 12) + (w1 << 6) + w2
                loc = plsc.bitcast(lin - qlo, jnp.uint32)
                m = loc < QB
                locc = plsc.bitcast(jnp.minimum(loc, QB - 1), jnp.int32)
                plsc.addupdate_scatter(hist, [locc], ones, mask=m)

        @pl.loop(0, NCH // 2)
        def _(g):
            for par in range(2):
                hist_chunk(g * 2 + par, par)

        pltpu.sync_copy(hist, counts_hbm.at[pair])
        xrow = pair * (QB // XCH)

        @pl.loop(0, QB // XCH)
        def _(t):
            pltpu.sync_copy(x_hbm.at[xrow + t], xbuf)

            @pl.loop(0, XCH // 16, unroll=4)
            def _(j):
                xv = xbuf[pl.ds(j * 16, 16)]
                hv = hist[pl.ds(t * XCH + j * 16, 16)]
                rbuf[pl.ds(j * 16, 16)] = jnp.where(hv > 0.0, xv, 0.0)

            pltpu.sync_copy(rbuf, r_hbm.at[xrow + t])


@jax.jit
def kernel(x, z):
    xf = x.reshape(B * NBINS // XCH, XCH)
    zt = jnp.transpose(z, (2, 0, 1))
    mesh = plsc.VectorSubcoreMesh(core_axis_name="c", subcore_axis_name="s")
    counts_f, r_f = pl.kernel(
        _sc_body,
        out_type=(
            jax.ShapeDtypeStruct((B * Q, QB), jnp.float32),
            jax.ShapeDtypeStruct((B * NBINS // XCH, XCH), jnp.float32),
        ),
        mesh=mesh,
        compiler_params=pltpu.CompilerParams(needs_layout_passes=False),
        scratch_types=[
            pltpu.VMEM((PCH,), jnp.float32),
            pltpu.VMEM((PCH,), jnp.float32),
            pltpu.VMEM((PCH,), jnp.float32),
            pltpu.VMEM((PCH,), jnp.float32),
            pltpu.VMEM((PCH,), jnp.float32),
            pltpu.VMEM((PCH,), jnp.float32),
            pltpu.VMEM((QB,), jnp.float32),
            pltpu.VMEM((XCH,), jnp.float32),
            pltpu.VMEM((XCH,), jnp.float32),
            pltpu.SemaphoreType.DMA,
            pltpu.SemaphoreType.DMA,
        ],
    )(zt, xf)
    return (counts_f.reshape(B, SH, SH, SH), r_f.reshape(B, SH, SH, SH))


# speedup vs baseline: 19.7786x; 1.0695x over previous
import jax
import jax.numpy as jnp
from jax import lax
from jax.experimental import pallas as pl
from jax.experimental.pallas import tpu as pltpu
from jax.experimental.pallas import tpu_sc as plsc

B = 16
N = 131072
SH = 64
NBINS = SH * SH * SH
Q = 4
QB = NBINS // Q
PCH = 4096
NCH = N // PCH
XCH = 8192

NC = 2
NS = 16
NW = NC * NS


def _sc_body(z_hbm, x_hbm, counts_hbm, r_hbm,
             za0, zb0, zc0, za1, zb1, zc1, hist, xbuf0, xbuf1, rbuf0, rbuf1,
             sem0, sem1, xsem0, xsem1, rsem0, rsem1, csem):
    c = lax.axis_index("c")
    s = lax.axis_index("s")
    wid = s * NC + c
    ones = jnp.ones((16,), jnp.float32)
    sems = (sem0, sem1)
    zbufs = ((za0, zb0, zc0), (za1, zb1, zc1))
    xbufs = (xbuf0, xbuf1)
    xsems = (xsem0, xsem1)
    rbufs = (rbuf0, rbuf1)
    rsems = (rsem0, rsem1)

    def z_copies(b, ch, par):
        return [
            pltpu.make_async_copy(
                z_hbm.at[k, b, pl.ds(ch * PCH, PCH)], zbufs[par][k], sems[par])
            for k in range(3)
        ]

    def z_start(b, ch, par):
        for cp in z_copies(b, ch, par):
            cp.start()

    def z_wait(b, ch, par):
        for cp in z_copies(b, ch, par):
            cp.wait()

    for rep in range(2):
        pair = wid + rep * NW
        b = pair // Q
        q = pair % Q
        qlo = q * QB

        z_start(b, 0, 0)

        @pl.loop(0, QB // 16, unroll=8)
        def _(i):
            hist[pl.ds(i * 16, 16)] = jnp.zeros((16,), jnp.float32)

        def hist_chunk(ch, par):
            z_wait(b, ch, par)

            @pl.when(ch + 1 < NCH)
            def _():
                z_start(b, ch + 1, 1 - par)

            zb = zbufs[par]

            @pl.loop(0, PCH // 16, unroll=4)
            def _(i):
                sl = pl.ds(i * 16, 16)
                v0 = zb[0][sl]
                v1 = zb[1][sl]
                v2 = zb[2][sl]
                w0 = ((v0 * 64.0) * 0.9999999).astype(jnp.int32)
                w1 = ((v1 * 64.0) * 0.9999999).astype(jnp.int32)
                w2 = ((v2 * 64.0) * 0.9999999).astype(jnp.int32)
                lin = (w0 << 12) + (w1 << 6) + w2
                loc = plsc.bitcast(lin - qlo, jnp.uint32)
                m = loc < QB
                locc = plsc.bitcast(jnp.minimum(loc, QB - 1), jnp.int32)
                plsc.addupdate_scatter(hist, [locc], ones, mask=m)

        @pl.loop(0, NCH // 2)
        def _(g):
            for par in range(2):
                hist_chunk(g * 2 + par, par)

        ccp = pltpu.make_async_copy(hist, counts_hbm.at[pair], csem)
        ccp.start()
        xrow = pair * (QB // XCH)

        def x_cp(t, par):
            return pltpu.make_async_copy(
                x_hbm.at[xrow + t], xbufs[par], xsems[par])

        x_cp(0, 0).start()

        def xr_chunk(t, par):
            x_cp(t, par).wait()

            @pl.when(t + 1 < QB // XCH)
            def _():
                x_cp(t + 1, 1 - par).start()

            @pl.when(t >= 2)
            def _():
                pltpu.make_async_copy(
                    rbufs[par], r_hbm.at[xrow + t - 2], rsems[par]).wait()

            xb = xbufs[par]
            rb = rbufs[par]

            @pl.loop(0, XCH // 16, unroll=4)
            def _(j):
                xv = xb[pl.ds(j * 16, 16)]
                hv = hist[pl.ds(t * XCH + j * 16, 16)]
                rb[pl.ds(j * 16, 16)] = jnp.where(hv > 0.0, xv, 0.0)

            pltpu.make_async_copy(rbufs[par], r_hbm.at[xrow + t], rsems[par]).start()

        @pl.loop(0, QB // (2 * XCH))
        def _(g):
            for par in range(2):
                xr_chunk(g * 2 + par, par)

        pltpu.make_async_copy(
            rbufs[0], r_hbm.at[xrow + QB // XCH - 2], rsems[0]).wait()
        pltpu.make_async_copy(
            rbufs[1], r_hbm.at[xrow + QB // XCH - 1], rsems[1]).wait()
        ccp.wait()


@jax.jit
def kernel(x, z):
    xf = x.reshape(B * NBINS // XCH, XCH)
    zt = jnp.transpose(z, (2, 0, 1))
    mesh = plsc.VectorSubcoreMesh(core_axis_name="c", subcore_axis_name="s")
    counts_f, r_f = pl.kernel(
        _sc_body,
        out_type=(
            jax.ShapeDtypeStruct((B * Q, QB), jnp.float32),
            jax.ShapeDtypeStruct((B * NBINS // XCH, XCH), jnp.float32),
        ),
        mesh=mesh,
        compiler_params=pltpu.CompilerParams(needs_layout_passes=False),
        scratch_types=[
            pltpu.VMEM((PCH,), jnp.float32),
            pltpu.VMEM((PCH,), jnp.float32),
            pltpu.VMEM((PCH,), jnp.float32),
            pltpu.VMEM((PCH,), jnp.float32),
            pltpu.VMEM((PCH,), jnp.float32),
            pltpu.VMEM((PCH,), jnp.float32),
            pltpu.VMEM((QB,), jnp.float32),
            pltpu.VMEM((XCH,), jnp.float32),
            pltpu.VMEM((XCH,), jnp.float32),
            pltpu.VMEM((XCH,), jnp.float32),
            pltpu.VMEM((XCH,), jnp.float32),
            pltpu.SemaphoreType.DMA,
            pltpu.SemaphoreType.DMA,
            pltpu.SemaphoreType.DMA,
            pltpu.SemaphoreType.DMA,
            pltpu.SemaphoreType.DMA,
            pltpu.SemaphoreType.DMA,
            pltpu.SemaphoreType.DMA,
        ],
    )(zt, xf)
    return (counts_f.reshape(B, SH, SH, SH), r_f.reshape(B, SH, SH, SH))
